# speedup vs baseline: 1.6708x; 1.6708x over previous
import functools

import jax
import jax.numpy as jnp
from jax import lax
from jax.experimental import pallas as pl
from jax.experimental.pallas import tpu as pltpu
from jax.experimental.pallas import tpu_sc as plsc

_B, _S, _D = 4, 2048, 128
_N = _B * _S
_L = 16
_NC, _NS = 2, 16
_NW = _NC * _NS
_BPW = _N // _NW

_mesh = plsc.VectorSubcoreMesh(core_axis_name="c", subcore_axis_name="s")


@functools.partial(
    pl.kernel,
    mesh=_mesh,
    out_type=jax.ShapeDtypeStruct((_N, _D), jnp.float32),
    scratch_types=[
        pltpu.VMEM((_BPW,), jnp.int32),
        pltpu.VMEM((_BPW,), jnp.int32),
        pltpu.VMEM((_BPW, _D), jnp.float32),
        pltpu.VMEM((_BPW, _D), jnp.float32),
        pltpu.SemaphoreType.DMA,
        pltpu.SemaphoreType.DMA,
    ],
)
def _embed(vt_hbm, pt_hbm, ids_hbm, pids_hbm, out_hbm,
           idx_v, pidx_v, rows_v, prows_v, sem_v, sem_p):
    wid = lax.axis_index("s") * _NC + lax.axis_index("c")
    base = wid * _BPW
    pltpu.sync_copy(ids_hbm.at[pl.ds(base, _BPW)], idx_v)
    pltpu.sync_copy(pids_hbm.at[pl.ds(base, _BPW)], pidx_v)
    cp_v = pltpu.async_copy(vt_hbm.at[idx_v], rows_v, sem_v)
    cp_p = pltpu.async_copy(pt_hbm.at[pidx_v], prows_v, sem_p)
    cp_v.wait()
    cp_p.wait()

    def body(i, carry):
        for j in range(_D // _L):
            s = pl.ds(j * _L, _L)
            rows_v[i, s] = rows_v[i, s] + prows_v[i, s]
        return carry

    lax.fori_loop(0, _BPW, body, 0)
    pltpu.sync_copy(rows_v, out_hbm.at[pl.ds(base, _BPW)])


def kernel(input_ids, position_ids, vocab_table, pos_table):
    ids = input_ids.reshape(-1).astype(jnp.int32)
    pids = position_ids.reshape(-1).astype(jnp.int32)
    out = _embed(vocab_table, pos_table, ids, pids)
    return out.reshape(_B, _S, _D)
